# initial kernel scaffold (unmeasured)
import jax
import jax.numpy as jnp
from jax import lax
from jax.experimental import pallas as pl
from jax.experimental.pallas import tpu as pltpu

N_DEV = 8
T = 1024
V_PER = 8192
D = 1024


def kernel(ids, E):
    def body(ids_ref, e_ref, out_ref, gat_ref, xb_ref, gsem, send_sem, recv_sem):
        my = lax.axis_index("i")
        base = my * V_PER

        def g_body(t, cnt):
            lid = ids_ref[t] - base
            owned = jnp.logical_and(lid >= 0, lid < V_PER)

            @pl.when(owned)
            def _():
                pltpu.make_async_copy(
                    e_ref.at[pl.ds(jnp.clip(lid, 0, V_PER - 1), 1), :],
                    gat_ref.at[pl.ds(t, 1), :],
                    gsem,
                ).start()

            return cnt + owned.astype(jnp.int32)

        own_cnt = lax.fori_loop(0, T, g_body, jnp.int32(0))
        pl.semaphore_wait(gsem, own_cnt)

        def s_body(t, _):
            lid = ids_ref[t] - base
            owned = jnp.logical_and(lid >= 0, lid < V_PER)

            @pl.when(owned)
            def _():
                xb_ref[pl.ds(t, 1), :] = gat_ref[pl.ds(t, 1), :].astype(
                    jnp.bfloat16
                )
                for j in range(1, N_DEV):
                    dst = lax.rem(my + j, N_DEV)
                    pltpu.make_async_remote_copy(
                        src_ref=xb_ref.at[pl.ds(t, 1), :],
                        dst_ref=xb_ref.at[pl.ds(t, 1), :],
                        send_sem=send_sem,
                        recv_sem=recv_sem,
                        device_id=(dst,),
                        device_id_type=pl.DeviceIdType.MESH,
                    ).start()

            return _

        lax.fori_loop(0, T, s_body, jnp.int32(0))

        pl.semaphore_wait(send_sem, own_cnt * (N_DEV - 1))
        pl.semaphore_wait(recv_sem, T - own_cnt)

        out_ref[:, :] = xb_ref[:, :].astype(jnp.float32)

    return pl.pallas_call(
        body,
        out_shape=jax.ShapeDtypeStruct((T, D), jnp.float32),
        in_specs=[
            pl.BlockSpec(memory_space=pltpu.SMEM),
            pl.BlockSpec(memory_space=pltpu.ANY),
        ],
        out_specs=pl.BlockSpec(memory_space=pltpu.VMEM),
        scratch_shapes=[
            pltpu.VMEM((T, D), jnp.float32),
            pltpu.VMEM((T, D), jnp.bfloat16),
            pltpu.SemaphoreType.DMA,
            pltpu.SemaphoreType.DMA,
            pltpu.SemaphoreType.DMA,
        ],
    )(ids, E)


# baseline (device time: 105415 ns/iter reference)
import jax
import jax.numpy as jnp
from jax import lax
from jax.experimental import pallas as pl
from jax.experimental.pallas import tpu as pltpu

N_DEV = 8
T = 1024
V_PER = 8192
D = 1024
R = 8
SEND_WINDOW = 16


def kernel(ids, E):
    def body(ids_ref, e_ref, out_ref, gat_ref, stage_ref, xb_ref, gsem,
             send_sem, recv_sem):
        my = lax.axis_index("i")
        base = my * V_PER

        barrier_sem = pltpu.get_barrier_semaphore()
        for j in range(1, N_DEV):
            pl.semaphore_signal(
                barrier_sem,
                1,
                device_id=(lax.rem(my + j, N_DEV),),
                device_id_type=pl.DeviceIdType.MESH,
            )
        pl.semaphore_wait(barrier_sem, N_DEV - 1)

        def g_body(t, _):
            lid = jnp.clip(ids_ref[t] - base, 0, V_PER - 1)
            pltpu.make_async_copy(
                e_ref.at[pl.ds(lid * R, R), :],
                gat_ref.at[pl.ds(t * R, R), :],
                gsem,
            ).start()
            return _

        lax.fori_loop(0, T, g_body, 0)
        pltpu.make_async_copy(
            e_ref.at[pl.ds(0, T * R), :], gat_ref.at[:, :], gsem
        ).wait()

        def s_body(t, cnt):
            lid = ids_ref[t] - base
            owned = jnp.logical_and(lid >= 0, lid < V_PER)

            @pl.when(owned)
            def _():
                stage_ref[pl.ds(t * R, R), :] = gat_ref[
                    pl.ds(t * R, R), :
                ].astype(jnp.bfloat16)
                for j in range(1, N_DEV):
                    dst = lax.rem(my + j, N_DEV)
                    pltpu.make_async_remote_copy(
                        src_ref=stage_ref.at[pl.ds(t * R, R), :],
                        dst_ref=xb_ref.at[pl.ds(t * R, R), :],
                        send_sem=send_sem,
                        recv_sem=recv_sem,
                        device_id=(dst,),
                        device_id_type=pl.DeviceIdType.MESH,
                    ).start()
                pltpu.make_async_copy(
                    stage_ref.at[pl.ds(t * R, R), :],
                    xb_ref.at[pl.ds(t * R, R), :],
                    recv_sem,
                ).start()

            new_cnt = cnt + owned.astype(jnp.int32)

            @pl.when(jnp.logical_and(owned, new_cnt > SEND_WINDOW))
            def _():
                pltpu.make_async_remote_copy(
                    src_ref=stage_ref.at[pl.ds(0, (N_DEV - 1) * R), :],
                    dst_ref=xb_ref.at[pl.ds(0, (N_DEV - 1) * R), :],
                    send_sem=send_sem,
                    recv_sem=recv_sem,
                    device_id=(my,),
                    device_id_type=pl.DeviceIdType.MESH,
                ).wait_send()

            return new_cnt

        own_cnt = lax.fori_loop(0, T, s_body, jnp.int32(0))

        def w_body(_, x):
            pltpu.make_async_remote_copy(
                src_ref=stage_ref.at[pl.ds(0, (N_DEV - 1) * R), :],
                dst_ref=xb_ref.at[pl.ds(0, (N_DEV - 1) * R), :],
                send_sem=send_sem,
                recv_sem=recv_sem,
                device_id=(my,),
                device_id_type=pl.DeviceIdType.MESH,
            ).wait_send()
            return x

        lax.fori_loop(0, jnp.minimum(own_cnt, SEND_WINDOW), w_body, 0)

        pltpu.make_async_remote_copy(
            src_ref=stage_ref.at[:, :],
            dst_ref=xb_ref.at[:, :],
            send_sem=send_sem,
            recv_sem=recv_sem,
            device_id=(my,),
            device_id_type=pl.DeviceIdType.MESH,
        ).wait_recv()

        out_ref[:, :] = xb_ref[:, :].astype(jnp.float32)

    out = pl.pallas_call(
        body,
        out_shape=jax.ShapeDtypeStruct((T * R, 128), jnp.float32),
        in_specs=[
            pl.BlockSpec(memory_space=pltpu.SMEM),
            pl.BlockSpec(memory_space=pl.ANY),
        ],
        out_specs=pl.BlockSpec(memory_space=pltpu.VMEM),
        scratch_shapes=[
            pltpu.VMEM((T * R, 128), jnp.float32),
            pltpu.VMEM((T * R, 128), jnp.bfloat16),
            pltpu.VMEM((T * R, 128), jnp.bfloat16),
            pltpu.SemaphoreType.DMA,
            pltpu.SemaphoreType.DMA,
            pltpu.SemaphoreType.DMA,
        ],
        compiler_params=pltpu.CompilerParams(collective_id=0),
    )(ids, E.reshape(V_PER * R, 128))
    return out.reshape(T, D)


# device time: 100224 ns/iter; 1.0518x vs baseline; 1.0518x over previous
import jax
import jax.numpy as jnp
from jax import lax
from jax.experimental import pallas as pl
from jax.experimental.pallas import tpu as pltpu

N_DEV = 8
T = 1024
V_PER = 8192
D = 1024
R = 8
SEND_WINDOW = 16


def kernel(ids, E):
    def body(ids_ref, e_ref, out_ref, gat_ref, stage_ref, xb_ref, gsem,
             send_sem, recv_sem):
        my = lax.axis_index("i")
        base = my * V_PER

        def g_body(t, _):
            lid = jnp.clip(ids_ref[t] - base, 0, V_PER - 1)
            pltpu.make_async_copy(
                e_ref.at[pl.ds(lid * R, R), :],
                gat_ref.at[pl.ds(t * R, R), :],
                gsem,
            ).start()
            return _

        lax.fori_loop(0, T, g_body, 0, unroll=8)

        barrier_sem = pltpu.get_barrier_semaphore()
        for j in range(1, N_DEV):
            pl.semaphore_signal(
                barrier_sem,
                1,
                device_id=(lax.rem(my + j, N_DEV),),
                device_id_type=pl.DeviceIdType.MESH,
            )
        pl.semaphore_wait(barrier_sem, N_DEV - 1)

        pltpu.make_async_copy(
            e_ref.at[pl.ds(0, T * R), :], gat_ref.at[:, :], gsem
        ).wait()
        stage_ref[:, :] = gat_ref[:, :].astype(jnp.bfloat16)

        def s_body(t, cnt):
            lid = ids_ref[t] - base
            owned = jnp.logical_and(lid >= 0, lid < V_PER)

            @pl.when(owned)
            def _():
                for j in range(1, N_DEV):
                    dst = lax.rem(my + j, N_DEV)
                    pltpu.make_async_remote_copy(
                        src_ref=stage_ref.at[pl.ds(t * R, R), :],
                        dst_ref=xb_ref.at[pl.ds(t * R, R), :],
                        send_sem=send_sem,
                        recv_sem=recv_sem,
                        device_id=(dst,),
                        device_id_type=pl.DeviceIdType.MESH,
                    ).start()
                pltpu.make_async_copy(
                    stage_ref.at[pl.ds(t * R, R), :],
                    xb_ref.at[pl.ds(t * R, R), :],
                    recv_sem,
                ).start()

            new_cnt = cnt + owned.astype(jnp.int32)

            @pl.when(jnp.logical_and(owned, new_cnt > SEND_WINDOW))
            def _():
                pltpu.make_async_remote_copy(
                    src_ref=stage_ref.at[pl.ds(0, (N_DEV - 1) * R), :],
                    dst_ref=xb_ref.at[pl.ds(0, (N_DEV - 1) * R), :],
                    send_sem=send_sem,
                    recv_sem=recv_sem,
                    device_id=(my,),
                    device_id_type=pl.DeviceIdType.MESH,
                ).wait_send()

            return new_cnt

        own_cnt = lax.fori_loop(0, T, s_body, jnp.int32(0), unroll=4)

        def w_body(_, x):
            pltpu.make_async_remote_copy(
                src_ref=stage_ref.at[pl.ds(0, (N_DEV - 1) * R), :],
                dst_ref=xb_ref.at[pl.ds(0, (N_DEV - 1) * R), :],
                send_sem=send_sem,
                recv_sem=recv_sem,
                device_id=(my,),
                device_id_type=pl.DeviceIdType.MESH,
            ).wait_send()
            return x

        lax.fori_loop(0, jnp.minimum(own_cnt, SEND_WINDOW), w_body, 0)

        pltpu.make_async_remote_copy(
            src_ref=stage_ref.at[:, :],
            dst_ref=xb_ref.at[:, :],
            send_sem=send_sem,
            recv_sem=recv_sem,
            device_id=(my,),
            device_id_type=pl.DeviceIdType.MESH,
        ).wait_recv()

        out_ref[:, :] = xb_ref[:, :]

    out = pl.pallas_call(
        body,
        out_shape=jax.ShapeDtypeStruct((T * R, 128), jnp.bfloat16),
        in_specs=[
            pl.BlockSpec(memory_space=pltpu.SMEM),
            pl.BlockSpec(memory_space=pl.ANY),
        ],
        out_specs=pl.BlockSpec(memory_space=pltpu.VMEM),
        scratch_shapes=[
            pltpu.VMEM((T * R, 128), jnp.float32),
            pltpu.VMEM((T * R, 128), jnp.bfloat16),
            pltpu.VMEM((T * R, 128), jnp.bfloat16),
            pltpu.SemaphoreType.DMA,
            pltpu.SemaphoreType.DMA,
            pltpu.SemaphoreType.DMA,
        ],
        compiler_params=pltpu.CompilerParams(collective_id=0),
    )(ids, E.reshape(V_PER * R, 128))
    return out.reshape(T, D)


# device time: 82379 ns/iter; 1.2796x vs baseline; 1.2166x over previous
import jax
import jax.numpy as jnp
from jax import lax
from jax.experimental import pallas as pl
from jax.experimental.pallas import tpu as pltpu

N_DEV = 8
T = 1024
V_PER = 8192
D = 1024
R = 8
SEND_WINDOW = 16


def kernel(ids, E):
    def body(ids_ref, e_ref, out_ref, gat_ref, stage_ref, xb_ref, gsem,
             send_sem, recv_sem):
        my = lax.axis_index("i")
        base = my * V_PER

        def g_body(t, cnt):
            lid = ids_ref[t] - base
            owned = jnp.logical_and(lid >= 0, lid < V_PER)

            @pl.when(owned)
            def _():
                pltpu.make_async_copy(
                    e_ref.at[pl.ds(jnp.clip(lid, 0, V_PER - 1) * R, R), :],
                    gat_ref.at[pl.ds(t * R, R), :],
                    gsem,
                ).start()

            return cnt + owned.astype(jnp.int32)

        own_cnt = lax.fori_loop(0, T, g_body, jnp.int32(0), unroll=8)

        barrier_sem = pltpu.get_barrier_semaphore()
        for j in range(1, N_DEV):
            pl.semaphore_signal(
                barrier_sem,
                1,
                device_id=(lax.rem(my + j, N_DEV),),
                device_id_type=pl.DeviceIdType.MESH,
            )
        pl.semaphore_wait(barrier_sem, N_DEV - 1)

        def gw_body(_, x):
            pltpu.make_async_copy(
                e_ref.at[pl.ds(0, R), :], gat_ref.at[pl.ds(0, R), :], gsem
            ).wait()
            return x

        lax.fori_loop(0, own_cnt, gw_body, 0)
        stage_ref[:, :] = gat_ref[:, :].astype(jnp.bfloat16)

        def s_body(t, cnt):
            lid = ids_ref[t] - base
            owned = jnp.logical_and(lid >= 0, lid < V_PER)

            @pl.when(owned)
            def _():
                for j in range(1, N_DEV):
                    dst = lax.rem(my + j, N_DEV)
                    pltpu.make_async_remote_copy(
                        src_ref=stage_ref.at[pl.ds(t * R, R), :],
                        dst_ref=xb_ref.at[pl.ds(t * R, R), :],
                        send_sem=send_sem,
                        recv_sem=recv_sem,
                        device_id=(dst,),
                        device_id_type=pl.DeviceIdType.MESH,
                    ).start()
                pltpu.make_async_copy(
                    stage_ref.at[pl.ds(t * R, R), :],
                    xb_ref.at[pl.ds(t * R, R), :],
                    recv_sem,
                ).start()

            new_cnt = cnt + owned.astype(jnp.int32)

            @pl.when(jnp.logical_and(owned, new_cnt > SEND_WINDOW))
            def _():
                pltpu.make_async_remote_copy(
                    src_ref=stage_ref.at[pl.ds(0, (N_DEV - 1) * R), :],
                    dst_ref=xb_ref.at[pl.ds(0, (N_DEV - 1) * R), :],
                    send_sem=send_sem,
                    recv_sem=recv_sem,
                    device_id=(my,),
                    device_id_type=pl.DeviceIdType.MESH,
                ).wait_send()

            return new_cnt

        own_cnt = lax.fori_loop(0, T, s_body, jnp.int32(0), unroll=4)

        def w_body(_, x):
            pltpu.make_async_remote_copy(
                src_ref=stage_ref.at[pl.ds(0, (N_DEV - 1) * R), :],
                dst_ref=xb_ref.at[pl.ds(0, (N_DEV - 1) * R), :],
                send_sem=send_sem,
                recv_sem=recv_sem,
                device_id=(my,),
                device_id_type=pl.DeviceIdType.MESH,
            ).wait_send()
            return x

        lax.fori_loop(0, jnp.minimum(own_cnt, SEND_WINDOW), w_body, 0)

        pltpu.make_async_remote_copy(
            src_ref=stage_ref.at[:, :],
            dst_ref=xb_ref.at[:, :],
            send_sem=send_sem,
            recv_sem=recv_sem,
            device_id=(my,),
            device_id_type=pl.DeviceIdType.MESH,
        ).wait_recv()

        out_ref[:, :] = xb_ref[:, :]

    out = pl.pallas_call(
        body,
        out_shape=jax.ShapeDtypeStruct((T * R, 128), jnp.bfloat16),
        in_specs=[
            pl.BlockSpec(memory_space=pltpu.SMEM),
            pl.BlockSpec(memory_space=pl.ANY),
        ],
        out_specs=pl.BlockSpec(memory_space=pltpu.VMEM),
        scratch_shapes=[
            pltpu.VMEM((T * R, 128), jnp.float32),
            pltpu.VMEM((T * R, 128), jnp.bfloat16),
            pltpu.VMEM((T * R, 128), jnp.bfloat16),
            pltpu.SemaphoreType.DMA,
            pltpu.SemaphoreType.DMA,
            pltpu.SemaphoreType.DMA,
        ],
        compiler_params=pltpu.CompilerParams(collective_id=0),
    )(ids, E.reshape(V_PER * R, 128))
    return out.reshape(T, D)


# device time: 81848 ns/iter; 1.2879x vs baseline; 1.0065x over previous
import jax
import jax.numpy as jnp
from jax import lax
from jax.experimental import pallas as pl
from jax.experimental.pallas import tpu as pltpu

N_DEV = 8
T = 1024
V_PER = 8192
D = 1024
R = 8
CH = 32
REG = 384


def kernel(ids, E):
    def body(ids_ref, e_ref, out_ref, gat_ref, stage_ref, recv_ref,
             cnt_ref, pos_ref, gsem, send_sem, recv_sem):
        my = lax.axis_index("i")
        base = my * V_PER

        for s in range(N_DEV):
            cnt_ref[s] = 0
            pos_ref[s] = 0

        def g_body(t, k):
            o = lax.shift_right_logical(ids_ref[t], 13)
            cnt_ref[o] = cnt_ref[o] + 1
            owned = o == my

            @pl.when(owned)
            def _():
                lid = ids_ref[t] - base
                pltpu.make_async_copy(
                    e_ref.at[pl.ds(lid * R, R), :],
                    gat_ref.at[pl.ds(k * R, R), :],
                    gsem,
                ).start()

            return k + owned.astype(jnp.int32)

        own_cnt = lax.fori_loop(0, T, g_body, jnp.int32(0), unroll=8)
        my_chunks = (own_cnt + CH - 1) // CH

        def tc_body(s, tot):
            return tot + (cnt_ref[s] + CH - 1) // CH

        total_chunks = lax.fori_loop(0, N_DEV, tc_body, jnp.int32(0))

        barrier_sem = pltpu.get_barrier_semaphore()
        for j in range(1, N_DEV):
            pl.semaphore_signal(
                barrier_sem,
                1,
                device_id=(lax.rem(my + j, N_DEV),),
                device_id_type=pl.DeviceIdType.MESH,
            )
        pl.semaphore_wait(barrier_sem, N_DEV - 1)

        def gw_body(_, x):
            pltpu.make_async_copy(
                e_ref.at[pl.ds(0, R), :], gat_ref.at[pl.ds(0, R), :], gsem
            ).wait()
            return x

        lax.fori_loop(0, own_cnt, gw_body, 0)
        stage_ref[:, :] = gat_ref[:, :].astype(jnp.bfloat16)

        def c_body(c, _):
            src = stage_ref.at[pl.ds(c * CH * R, CH * R), :]
            dst = recv_ref.at[pl.ds((my * REG + c * CH) * R, CH * R), :]
            for j in range(1, N_DEV):
                pltpu.make_async_remote_copy(
                    src_ref=src,
                    dst_ref=dst,
                    send_sem=send_sem,
                    recv_sem=recv_sem,
                    device_id=(lax.rem(my + j, N_DEV),),
                    device_id_type=pl.DeviceIdType.MESH,
                ).start()
            pltpu.make_async_copy(src, dst, recv_sem).start()
            return _

        lax.fori_loop(0, my_chunks, c_body, 0)

        def w_body(_, x):
            pltpu.make_async_remote_copy(
                src_ref=stage_ref.at[pl.ds(0, (N_DEV - 1) * CH * R), :],
                dst_ref=recv_ref.at[pl.ds(0, (N_DEV - 1) * CH * R), :],
                send_sem=send_sem,
                recv_sem=recv_sem,
                device_id=(my,),
                device_id_type=pl.DeviceIdType.MESH,
            ).wait_send()
            return x

        lax.fori_loop(0, my_chunks, w_body, 0)

        def r_body(_, x):
            pltpu.make_async_remote_copy(
                src_ref=stage_ref.at[pl.ds(0, CH * R), :],
                dst_ref=recv_ref.at[pl.ds(0, CH * R), :],
                send_sem=send_sem,
                recv_sem=recv_sem,
                device_id=(my,),
                device_id_type=pl.DeviceIdType.MESH,
            ).wait_recv()
            return x

        lax.fori_loop(0, total_chunks, r_body, 0)

        def u_body(t, _):
            o = lax.shift_right_logical(ids_ref[t], 13)
            k = pos_ref[o]
            pos_ref[o] = k + 1
            out_ref[pl.ds(t * R, R), :] = recv_ref[
                pl.ds((o * REG + k) * R, R), :
            ]
            return _

        lax.fori_loop(0, T, u_body, 0, unroll=8)

    out = pl.pallas_call(
        body,
        out_shape=jax.ShapeDtypeStruct((T * R, 128), jnp.bfloat16),
        in_specs=[
            pl.BlockSpec(memory_space=pltpu.SMEM),
            pl.BlockSpec(memory_space=pl.ANY),
        ],
        out_specs=pl.BlockSpec(memory_space=pltpu.VMEM),
        scratch_shapes=[
            pltpu.VMEM((T * R, 128), jnp.float32),
            pltpu.VMEM((T * R, 128), jnp.bfloat16),
            pltpu.VMEM((N_DEV * REG * R, 128), jnp.bfloat16),
            pltpu.SMEM((N_DEV,), jnp.int32),
            pltpu.SMEM((N_DEV,), jnp.int32),
            pltpu.SemaphoreType.DMA,
            pltpu.SemaphoreType.DMA,
            pltpu.SemaphoreType.DMA,
        ],
        compiler_params=pltpu.CompilerParams(collective_id=0),
    )(ids, E.reshape(V_PER * R, 128))
    return out.reshape(T, D)


# device time: 73387 ns/iter; 1.4364x vs baseline; 1.1153x over previous
import jax
import jax.numpy as jnp
from jax import lax
from jax.experimental import pallas as pl
from jax.experimental.pallas import tpu as pltpu

N_DEV = 8
T = 1024
V_PER = 8192
D = 1024
R = 8
SEND_WINDOW = 16


def kernel(ids, E):
    owner = lax.shift_right_logical(ids, 13)
    perm = jnp.argsort(owner, stable=True).astype(jnp.int32)
    sorted_ids = ids[perm]
    counts = jnp.sum(
        owner[:, None] == jnp.arange(N_DEV, dtype=ids.dtype)[None, :],
        axis=0,
        dtype=jnp.int32,
    )
    starts = jnp.concatenate(
        [jnp.zeros((1,), jnp.int32), jnp.cumsum(counts)[:-1].astype(jnp.int32)]
    )

    def body(perm_ref, sid_ref, cnt_ref, start_ref, e_ref, out_ref,
             gat_ref, stage_ref, xb_ref, gsem, send_sem, recv_sem):
        my = lax.axis_index("i")
        base = my * V_PER
        my_start = start_ref[my]
        own_cnt = cnt_ref[my]

        def g_body(k, _):
            lid = sid_ref[my_start + k] - base
            pltpu.make_async_copy(
                e_ref.at[pl.ds(lid * R, R), :],
                gat_ref.at[pl.ds(k * R, R), :],
                gsem,
            ).start()
            return _

        lax.fori_loop(0, own_cnt, g_body, 0)

        barrier_sem = pltpu.get_barrier_semaphore()
        for j in range(1, N_DEV):
            pl.semaphore_signal(
                barrier_sem,
                1,
                device_id=(lax.rem(my + j, N_DEV),),
                device_id_type=pl.DeviceIdType.MESH,
            )
        pl.semaphore_wait(barrier_sem, N_DEV - 1)

        def gw_body(_, x):
            pltpu.make_async_copy(
                e_ref.at[pl.ds(0, R), :], gat_ref.at[pl.ds(0, R), :], gsem
            ).wait()
            return x

        lax.fori_loop(0, own_cnt, gw_body, 0)
        stage_ref[:, :] = gat_ref[:, :].astype(jnp.bfloat16)

        def s_body(k, carry):
            t = perm_ref[my_start + k]
            src = stage_ref.at[pl.ds(k * R, R), :]
            for j in range(1, N_DEV):
                pltpu.make_async_remote_copy(
                    src_ref=src,
                    dst_ref=xb_ref.at[pl.ds(t * R, R), :],
                    send_sem=send_sem,
                    recv_sem=recv_sem,
                    device_id=(lax.rem(my + j, N_DEV),),
                    device_id_type=pl.DeviceIdType.MESH,
                ).start()
            pltpu.make_async_copy(
                src, xb_ref.at[pl.ds(t * R, R), :], recv_sem
            ).start()

            @pl.when(k >= SEND_WINDOW)
            def _():
                pltpu.make_async_remote_copy(
                    src_ref=stage_ref.at[pl.ds(0, (N_DEV - 1) * R), :],
                    dst_ref=xb_ref.at[pl.ds(0, (N_DEV - 1) * R), :],
                    send_sem=send_sem,
                    recv_sem=recv_sem,
                    device_id=(my,),
                    device_id_type=pl.DeviceIdType.MESH,
                ).wait_send()

            return carry

        lax.fori_loop(0, own_cnt, s_body, 0)

        def w_body(_, x):
            pltpu.make_async_remote_copy(
                src_ref=stage_ref.at[pl.ds(0, (N_DEV - 1) * R), :],
                dst_ref=xb_ref.at[pl.ds(0, (N_DEV - 1) * R), :],
                send_sem=send_sem,
                recv_sem=recv_sem,
                device_id=(my,),
                device_id_type=pl.DeviceIdType.MESH,
            ).wait_send()
            return x

        lax.fori_loop(0, jnp.minimum(own_cnt, SEND_WINDOW), w_body, 0)

        pltpu.make_async_remote_copy(
            src_ref=stage_ref.at[:, :],
            dst_ref=xb_ref.at[:, :],
            send_sem=send_sem,
            recv_sem=recv_sem,
            device_id=(my,),
            device_id_type=pl.DeviceIdType.MESH,
        ).wait_recv()

        out_ref[:, :] = xb_ref[:, :]

    out = pl.pallas_call(
        body,
        out_shape=jax.ShapeDtypeStruct((T * R, 128), jnp.bfloat16),
        in_specs=[
            pl.BlockSpec(memory_space=pltpu.SMEM),
            pl.BlockSpec(memory_space=pltpu.SMEM),
            pl.BlockSpec(memory_space=pltpu.SMEM),
            pl.BlockSpec(memory_space=pltpu.SMEM),
            pl.BlockSpec(memory_space=pl.ANY),
        ],
        out_specs=pl.BlockSpec(memory_space=pltpu.VMEM),
        scratch_shapes=[
            pltpu.VMEM((T * R, 128), jnp.float32),
            pltpu.VMEM((T * R, 128), jnp.bfloat16),
            pltpu.VMEM((T * R, 128), jnp.bfloat16),
            pltpu.SemaphoreType.DMA,
            pltpu.SemaphoreType.DMA,
            pltpu.SemaphoreType.DMA,
        ],
        compiler_params=pltpu.CompilerParams(collective_id=0),
    )(perm, sorted_ids, counts, starts, E.reshape(V_PER * R, 128))
    return out.reshape(T, D)


# device time: 66483 ns/iter; 1.5856x vs baseline; 1.1038x over previous
import jax
import jax.numpy as jnp
from jax import lax
from jax.experimental import pallas as pl
from jax.experimental.pallas import tpu as pltpu

N_DEV = 8
T = 1024
V_PER = 8192
D = 1024
R = 8
SEND_WINDOW = 16


def kernel(ids, E):
    owner = lax.shift_right_logical(ids, 13)
    perm = jnp.argsort(owner, stable=True).astype(jnp.int32)
    counts = jnp.sum(
        owner[:, None] == jnp.arange(N_DEV, dtype=ids.dtype)[None, :],
        axis=0,
        dtype=jnp.int32,
    )
    starts = jnp.concatenate(
        [jnp.zeros((1,), jnp.int32), jnp.cumsum(counts)[:-1].astype(jnp.int32)]
    )

    def body(perm_ref, ids_ref, cnt_ref, start_ref, e_ref, out_ref,
             gat_ref, stage_ref, xb_ref, gsem, send_sem, recv_sem):
        my = lax.axis_index("i")
        base = my * V_PER
        my_start = start_ref[my]
        own_cnt = cnt_ref[my]

        def g_body(k, _):
            lid = ids_ref[perm_ref[my_start + k]] - base
            pltpu.make_async_copy(
                e_ref.at[pl.ds(lid * R, R), :],
                gat_ref.at[pl.ds(k * R, R), :],
                gsem,
            ).start()
            return _

        lax.fori_loop(0, own_cnt, g_body, 0)

        barrier_sem = pltpu.get_barrier_semaphore()
        for j in range(1, N_DEV):
            pl.semaphore_signal(
                barrier_sem,
                1,
                device_id=(lax.rem(my + j, N_DEV),),
                device_id_type=pl.DeviceIdType.MESH,
            )
        pl.semaphore_wait(barrier_sem, N_DEV - 1)

        def gw_body(_, x):
            pltpu.make_async_copy(
                e_ref.at[pl.ds(0, R), :], gat_ref.at[pl.ds(0, R), :], gsem
            ).wait()
            return x

        lax.fori_loop(0, own_cnt, gw_body, 0)
        stage_ref[:, :] = gat_ref[:, :].astype(jnp.bfloat16)

        def s_body(k, carry):
            t = perm_ref[my_start + k]
            src = stage_ref.at[pl.ds(k * R, R), :]
            for j in range(1, N_DEV):
                pltpu.make_async_remote_copy(
                    src_ref=src,
                    dst_ref=xb_ref.at[pl.ds(t * R, R), :],
                    send_sem=send_sem,
                    recv_sem=recv_sem,
                    device_id=(lax.rem(my + j, N_DEV),),
                    device_id_type=pl.DeviceIdType.MESH,
                ).start()
            pltpu.make_async_copy(
                src, xb_ref.at[pl.ds(t * R, R), :], recv_sem
            ).start()

            @pl.when(k >= SEND_WINDOW)
            def _():
                pltpu.make_async_remote_copy(
                    src_ref=stage_ref.at[pl.ds(0, (N_DEV - 1) * R), :],
                    dst_ref=xb_ref.at[pl.ds(0, (N_DEV - 1) * R), :],
                    send_sem=send_sem,
                    recv_sem=recv_sem,
                    device_id=(my,),
                    device_id_type=pl.DeviceIdType.MESH,
                ).wait_send()

            return carry

        lax.fori_loop(0, own_cnt, s_body, 0)

        def w_body(_, x):
            pltpu.make_async_remote_copy(
                src_ref=stage_ref.at[pl.ds(0, (N_DEV - 1) * R), :],
                dst_ref=xb_ref.at[pl.ds(0, (N_DEV - 1) * R), :],
                send_sem=send_sem,
                recv_sem=recv_sem,
                device_id=(my,),
                device_id_type=pl.DeviceIdType.MESH,
            ).wait_send()
            return x

        lax.fori_loop(0, jnp.minimum(own_cnt, SEND_WINDOW), w_body, 0)

        pltpu.make_async_remote_copy(
            src_ref=stage_ref.at[:, :],
            dst_ref=xb_ref.at[:, :],
            send_sem=send_sem,
            recv_sem=recv_sem,
            device_id=(my,),
            device_id_type=pl.DeviceIdType.MESH,
        ).wait_recv()

        out_ref[:, :] = xb_ref[:, :]

    out = pl.pallas_call(
        body,
        out_shape=jax.ShapeDtypeStruct((T * R, 128), jnp.bfloat16),
        in_specs=[
            pl.BlockSpec(memory_space=pltpu.SMEM),
            pl.BlockSpec(memory_space=pltpu.SMEM),
            pl.BlockSpec(memory_space=pltpu.SMEM),
            pl.BlockSpec(memory_space=pltpu.SMEM),
            pl.BlockSpec(memory_space=pl.ANY),
        ],
        out_specs=pl.BlockSpec(memory_space=pltpu.VMEM),
        scratch_shapes=[
            pltpu.VMEM((T * R, 128), jnp.float32),
            pltpu.VMEM((T * R, 128), jnp.bfloat16),
            pltpu.VMEM((T * R, 128), jnp.bfloat16),
            pltpu.SemaphoreType.DMA,
            pltpu.SemaphoreType.DMA,
            pltpu.SemaphoreType.DMA,
        ],
        compiler_params=pltpu.CompilerParams(collective_id=0),
    )(perm, ids, counts, starts, E.reshape(V_PER * R, 128))
    return out.reshape(T, D)


# device time: 66271 ns/iter; 1.5907x vs baseline; 1.0032x over previous
import jax
import jax.numpy as jnp
from jax import lax
from jax.experimental import pallas as pl
from jax.experimental.pallas import tpu as pltpu

N_DEV = 8
T = 1024
V_PER = 8192
D = 1024
R = 8
SEND_WINDOW = 64


def kernel(ids, E):
    owner = lax.shift_right_logical(ids, 13)
    perm = jnp.argsort(owner, stable=True).astype(jnp.int32)
    counts = jnp.sum(
        owner[:, None] == jnp.arange(N_DEV, dtype=ids.dtype)[None, :],
        axis=0,
        dtype=jnp.int32,
    )
    starts = jnp.concatenate(
        [jnp.zeros((1,), jnp.int32), jnp.cumsum(counts)[:-1].astype(jnp.int32)]
    )

    def body(perm_ref, ids_ref, cnt_ref, start_ref, e_ref, out_ref,
             gat_ref, stage_ref, xb_ref, gsem, send_sem, recv_sem):
        my = lax.axis_index("i")
        base = my * V_PER
        my_start = start_ref[my]
        own_cnt = cnt_ref[my]

        def g_body(k, _):
            lid = ids_ref[perm_ref[my_start + k]] - base
            pltpu.make_async_copy(
                e_ref.at[pl.ds(lid * R, R), :],
                gat_ref.at[pl.ds(k * R, R), :],
                gsem,
            ).start()
            return _

        lax.fori_loop(0, own_cnt, g_body, 0)

        barrier_sem = pltpu.get_barrier_semaphore()
        for j in range(1, N_DEV):
            pl.semaphore_signal(
                barrier_sem,
                1,
                device_id=(lax.rem(my + j, N_DEV),),
                device_id_type=pl.DeviceIdType.MESH,
            )
        pl.semaphore_wait(barrier_sem, N_DEV - 1)

        def gw_body(_, x):
            pltpu.make_async_copy(
                e_ref.at[pl.ds(0, R), :], gat_ref.at[pl.ds(0, R), :], gsem
            ).wait()
            return x

        lax.fori_loop(0, own_cnt, gw_body, 0)
        stage_ref[:, :] = gat_ref[:, :].astype(jnp.bfloat16)

        def s_body(k, carry):
            t = perm_ref[my_start + k]
            src = stage_ref.at[pl.ds(k * R, R), :]
            for j in range(1, N_DEV):
                pltpu.make_async_remote_copy(
                    src_ref=src,
                    dst_ref=xb_ref.at[pl.ds(t * R, R), :],
                    send_sem=send_sem,
                    recv_sem=recv_sem,
                    device_id=(lax.rem(my + j, N_DEV),),
                    device_id_type=pl.DeviceIdType.MESH,
                ).start()
            pltpu.make_async_copy(
                src, xb_ref.at[pl.ds(t * R, R), :], recv_sem
            ).start()

            @pl.when(k >= SEND_WINDOW)
            def _():
                pltpu.make_async_remote_copy(
                    src_ref=stage_ref.at[pl.ds(0, (N_DEV - 1) * R), :],
                    dst_ref=xb_ref.at[pl.ds(0, (N_DEV - 1) * R), :],
                    send_sem=send_sem,
                    recv_sem=recv_sem,
                    device_id=(my,),
                    device_id_type=pl.DeviceIdType.MESH,
                ).wait_send()

            return carry

        lax.fori_loop(0, own_cnt, s_body, 0)

        def w_body(_, x):
            pltpu.make_async_remote_copy(
                src_ref=stage_ref.at[pl.ds(0, (N_DEV - 1) * R), :],
                dst_ref=xb_ref.at[pl.ds(0, (N_DEV - 1) * R), :],
                send_sem=send_sem,
                recv_sem=recv_sem,
                device_id=(my,),
                device_id_type=pl.DeviceIdType.MESH,
            ).wait_send()
            return x

        lax.fori_loop(0, jnp.minimum(own_cnt, SEND_WINDOW), w_body, 0)

        pltpu.make_async_remote_copy(
            src_ref=stage_ref.at[:, :],
            dst_ref=xb_ref.at[:, :],
            send_sem=send_sem,
            recv_sem=recv_sem,
            device_id=(my,),
            device_id_type=pl.DeviceIdType.MESH,
        ).wait_recv()

        out_ref[:, :] = xb_ref[:, :]

    out = pl.pallas_call(
        body,
        out_shape=jax.ShapeDtypeStruct((T * R, 128), jnp.bfloat16),
        in_specs=[
            pl.BlockSpec(memory_space=pltpu.SMEM),
            pl.BlockSpec(memory_space=pltpu.SMEM),
            pl.BlockSpec(memory_space=pltpu.SMEM),
            pl.BlockSpec(memory_space=pltpu.SMEM),
            pl.BlockSpec(memory_space=pl.ANY),
        ],
        out_specs=pl.BlockSpec(memory_space=pltpu.VMEM),
        scratch_shapes=[
            pltpu.VMEM((T * R, 128), jnp.float32),
            pltpu.VMEM((T * R, 128), jnp.bfloat16),
            pltpu.VMEM((T * R, 128), jnp.bfloat16),
            pltpu.SemaphoreType.DMA,
            pltpu.SemaphoreType.DMA,
            pltpu.SemaphoreType.DMA,
        ],
        compiler_params=pltpu.CompilerParams(collective_id=0),
    )(perm, ids, counts, starts, E.reshape(V_PER * R, 128))
    return out.reshape(T, D)
